# baseline (speedup 1.0000x reference)
import dataclasses
import functools

import jax
import jax.numpy as jnp
from jax import lax
from jax.experimental import pallas as pl
from jax.experimental.pallas import tpu as pltpu
from jax.experimental.pallas import tpu_sc as plsc

_NC = 2
_NS = 16
_NW = _NC * _NS
_LANES = 16


def _make_spmm(n, e, d, tc_tiling=True):
  epw = e // _NW
  chunk = 80
  nchunk = epw // chunk
  ngrp = 5
  grp = nchunk // ngrp
  rps = 624
  tail = n - rps * _NS
  zrows = 16
  assert epw % chunk == 0 and rps % zrows == 0 and 0 <= tail <= zrows
  assert nchunk % ngrp == 0
  mesh = plsc.VectorSubcoreMesh(core_axis_name="c", subcore_axis_name="s")
  cp = pltpu.CompilerParams()
  if "needs_layout_passes" in pltpu.CompilerParams.__dataclass_fields__:
    cp = dataclasses.replace(cp, needs_layout_passes=False)
  if not tc_tiling:
    cp = dataclasses.replace(cp, use_tc_tiling_on_sc=False)

  @functools.partial(
      pl.kernel,
      compiler_params=cp,
      out_type=jax.ShapeDtypeStruct((_NC, n, d), jnp.float32),
      mesh=mesh,
      scratch_types=[
          pltpu.VMEM((grp, chunk), jnp.int32),
          pltpu.VMEM((grp, chunk), jnp.int32),
          pltpu.VMEM((grp * chunk,), jnp.float32),
          pltpu.VMEM((chunk, d), jnp.float32),
          pltpu.VMEM((chunk, d), jnp.float32),
          pltpu.VMEM((chunk, d), jnp.float32),
          pltpu.VMEM_SHARED((n, d), jnp.float32),
          [pltpu.SemaphoreType.DMA] * 3,
          [pltpu.SemaphoreType.DMA] * 3,
      ],
  )
  def spmm(h_hbm, row_hbm, col_hbm, w_hbm, out_hbm,
           row_v, col_v, w_v, buf, buf1, buf2, acc, gsems, ssems):
    cid = lax.axis_index("c")
    sid = lax.axis_index("s")
    wid = sid * _NC + cid

    zero = jnp.zeros((_LANES,), jnp.float32)

    @pl.loop(0, zrows)
    def _(i):
      for k in range(d // _LANES):
        buf[i, pl.ds(k * _LANES, _LANES)] = zero

    @pl.loop(0, rps // zrows)
    def _(i):
      pltpu.sync_copy(buf.at[pl.ds(0, zrows)],
                      acc.at[pl.ds(sid * rps + i * zrows, zrows)])

    @pl.when(sid == _NS - 1)
    def _():
      pltpu.sync_copy(buf.at[pl.ds(0, tail)],
                      acc.at[pl.ds(_NS * rps, tail)])

    plsc.subcore_barrier()

    def scale(bufref, j):
      dnums = lax.GatherDimensionNumbers(
          offset_dims=(), collapsed_slice_dims=(0,), start_index_map=(0,))

      @plsc.parallel_loop(0, chunk, step=_LANES, unroll=2)
      def _(e0):
        w16 = w_v[pl.ds(j * chunk + e0, _LANES)]
        for r in range(_LANES):
          wreg = lax.gather(
              w16, jnp.full((_LANES, 1), r, jnp.int32), dnums, (1,),
              mode=lax.GatherScatterMode.PROMISE_IN_BOUNDS)
          for k in range(d // _LANES):
            sl = (e0 + r, pl.ds(k * _LANES, _LANES))
            bufref[sl] = bufref[sl] * wreg

    bufs = (buf, buf1, buf2)

    def wait_gather(b, j):
      pltpu.make_async_copy(h_hbm.at[col_v.at[j]], bufs[b], gsems[b]).wait()

    def wait_scatter(b):
      pltpu.make_async_copy(bufs[b], acc.at[row_v.at[0]], ssems[b]).wait()

    @pl.loop(0, ngrp)
    def _(g):
      @pl.when(g > 0)
      def _():
        for b in range(3):
          wait_scatter(b)

      pltpu.sync_copy(row_hbm.at[wid, g], row_v)
      pltpu.sync_copy(col_hbm.at[wid, g], col_v)
      pltpu.sync_copy(w_hbm.at[wid, g], w_v)

      pltpu.async_copy(h_hbm.at[col_v.at[0]], buf, gsems[0])
      pltpu.async_copy(h_hbm.at[col_v.at[1]], buf1, gsems[1])

      @pl.loop(0, (grp + 2) // 3)
      def _(i):
        for b in range(3):
          j = 3 * i + b

          @pl.when(j < grp)
          def _():
            wait_gather(b, j)
            b2 = (b + 2) % 3

            @pl.when(j + 2 < grp)
            def _():
              @pl.when(j >= 1)
              def _():
                wait_scatter(b2)
              pltpu.async_copy(h_hbm.at[col_v.at[j + 2]], bufs[b2], gsems[b2])

            scale(bufs[b], j)
            pltpu.async_copy(bufs[b], acc.at[row_v.at[j]], ssems[b], add=True)

    for b in range(3):
      wait_scatter(b)

    plsc.subcore_barrier()

    pltpu.sync_copy(acc.at[pl.ds(sid * rps, rps)],
                    out_hbm.at[cid, pl.ds(sid * rps, rps)])

    @pl.when(sid == _NS - 1)
    def _():
      pltpu.sync_copy(acc.at[pl.ds(_NS * rps, tail)],
                      out_hbm.at[cid, pl.ds(_NS * rps, tail)])

  return spmm


def _make_spmm_bf16(n, e, d):
  epw = e // _NW
  chunk = 80
  nchunk = epw // chunk
  ngrp = 5
  grp = nchunk // ngrp
  rps = 624
  tail = n - rps * _NS
  zrows = 16
  assert nchunk % ngrp == 0 and d % 32 == 0
  mesh = plsc.VectorSubcoreMesh(core_axis_name="c", subcore_axis_name="s")
  cp = pltpu.CompilerParams()
  if "needs_layout_passes" in pltpu.CompilerParams.__dataclass_fields__:
    cp = dataclasses.replace(cp, needs_layout_passes=False)
  cp = dataclasses.replace(cp, use_tc_tiling_on_sc=False)

  @functools.partial(
      pl.kernel,
      compiler_params=cp,
      out_type=jax.ShapeDtypeStruct((_NC, n, d), jnp.float32),
      mesh=mesh,
      scratch_types=[
          pltpu.VMEM((grp, chunk), jnp.int32),
          pltpu.VMEM((grp, chunk), jnp.int32),
          pltpu.VMEM((grp * chunk,), jnp.float32),
          pltpu.VMEM((chunk, d), jnp.bfloat16),
          pltpu.VMEM((chunk, d), jnp.bfloat16),
          pltpu.VMEM((chunk, d), jnp.float32),
          pltpu.VMEM((chunk, d), jnp.float32),
          pltpu.VMEM_SHARED((n, d), jnp.float32),
          [pltpu.SemaphoreType.DMA] * 2,
          [pltpu.SemaphoreType.DMA] * 2,
      ],
  )
  def spmm(h_hbm, row_hbm, col_hbm, w_hbm, out_hbm,
           row_v, col_v, w_v, gb0, gb1, sb0, sb1, acc, gsems, ssems):
    cid = lax.axis_index("c")
    sid = lax.axis_index("s")
    wid = sid * _NC + cid
    gbufs = (gb0, gb1)
    sbufs = (sb0, sb1)

    zero = jnp.zeros((_LANES,), jnp.float32)

    @pl.loop(0, zrows)
    def _(i):
      for k in range(d // _LANES):
        sb0[i, pl.ds(k * _LANES, _LANES)] = zero

    @pl.loop(0, rps // zrows)
    def _(i):
      pltpu.sync_copy(sb0.at[pl.ds(0, zrows)],
                      acc.at[pl.ds(sid * rps + i * zrows, zrows)])

    @pl.when(sid == _NS - 1)
    def _():
      pltpu.sync_copy(sb0.at[pl.ds(0, tail)],
                      acc.at[pl.ds(_NS * rps, tail)])

    plsc.subcore_barrier()

    mask = jnp.full((_LANES,), -65536, jnp.int32)

    def scale(gbuf, sbuf, j):
      dnums = lax.GatherDimensionNumbers(
          offset_dims=(), collapsed_slice_dims=(0,), start_index_map=(0,))

      @plsc.parallel_loop(0, chunk, step=_LANES, unroll=2)
      def _(e0):
        w16 = w_v[pl.ds(j * chunk + e0, _LANES)]
        for r in range(_LANES):
          wreg = lax.gather(
              w16, jnp.full((_LANES, 1), r, jnp.int32), dnums, (1,),
              mode=lax.GatherScatterMode.PROMISE_IN_BOUNDS)
          for k in range(d // 32):
            v = gbuf[e0 + r, pl.ds(32 * k, 32)]
            vi = plsc.bitcast(v, jnp.int32)
            lo = plsc.bitcast(lax.shift_left(vi, 16), jnp.float32)
            hi = plsc.bitcast(vi & mask, jnp.float32)
            sbuf[e0 + r, pl.ds(32 * k, _LANES)] = lo * wreg
            sbuf[e0 + r, pl.ds(32 * k + _LANES, _LANES)] = hi * wreg

    def wait_gather(b, j):
      pltpu.make_async_copy(h_hbm.at[col_v.at[j]], gbufs[b], gsems[b]).wait()

    def wait_scatter(b):
      pltpu.make_async_copy(sbufs[b], acc.at[row_v.at[0]], ssems[b]).wait()

    @pl.loop(0, ngrp)
    def _(g):
      @pl.when(g > 0)
      def _():
        for b in range(2):
          wait_scatter(b)

      pltpu.sync_copy(row_hbm.at[wid, g], row_v)
      pltpu.sync_copy(col_hbm.at[wid, g], col_v)
      pltpu.sync_copy(w_hbm.at[wid, g], w_v)

      pltpu.async_copy(h_hbm.at[col_v.at[0]], gb0, gsems[0])

      @pl.loop(0, (grp + 1) // 2)
      def _(i):
        for b in range(2):
          j = 2 * i + b

          @pl.when(j < grp)
          def _():
            @pl.when(j + 1 < grp)
            def _():
              pltpu.async_copy(h_hbm.at[col_v.at[j + 1]],
                               gbufs[1 - b], gsems[1 - b])
            wait_gather(b, j)

            @pl.when(j >= 2)
            def _():
              wait_scatter(b)

            scale(gbufs[b], sbufs[b], j)
            pltpu.async_copy(sbufs[b], acc.at[row_v.at[j]], ssems[b], add=True)

    for b in range(2):
      wait_scatter(b)

    plsc.subcore_barrier()

    pltpu.sync_copy(acc.at[pl.ds(sid * rps, rps)],
                    out_hbm.at[cid, pl.ds(sid * rps, rps)])

    @pl.when(sid == _NS - 1)
    def _():
      pltpu.sync_copy(acc.at[pl.ds(_NS * rps, tail)],
                      out_hbm.at[cid, pl.ds(_NS * rps, tail)])

  return spmm


def _interleave_bf16(h):
  n, d = h.shape
  return (h.reshape(n, d // 32, 2, _LANES).swapaxes(2, 3)
          .reshape(n, d).astype(jnp.bfloat16))


def _mm(x, w, bm):
  n, k = x.shape
  m = w.shape[1]

  def body(x_ref, w_ref, o_ref):
    o_ref[...] = jnp.dot(x_ref[...], w_ref[...],
                         preferred_element_type=jnp.float32)

  return pl.pallas_call(
      body,
      grid=(n // bm,),
      in_specs=[pl.BlockSpec((bm, k), lambda i: (i, 0)),
                pl.BlockSpec((k, m), lambda i: (0, 0))],
      out_specs=pl.BlockSpec((bm, m), lambda i: (i, 0)),
      out_shape=jax.ShapeDtypeStruct((n, m), jnp.float32),
  )(x, w)


def _add_mm_relu_mm(p, w1, w2, bm):
  _, n, k = p.shape
  m = w2.shape[1]

  def body(p_ref, w1_ref, w2_ref, o_ref):
    t = jnp.dot(p_ref[0] + p_ref[1], w1_ref[...],
                preferred_element_type=jnp.float32)
    o_ref[...] = jnp.dot(jnp.maximum(t, 0.0), w2_ref[...],
                         preferred_element_type=jnp.float32)

  return pl.pallas_call(
      body,
      grid=(n // bm,),
      in_specs=[pl.BlockSpec((2, bm, k), lambda i: (0, i, 0)),
                pl.BlockSpec((k, w1.shape[1]), lambda i: (0, 0)),
                pl.BlockSpec((w2.shape[0], m), lambda i: (0, 0))],
      out_specs=pl.BlockSpec((bm, m), lambda i: (i, 0)),
      out_shape=jax.ShapeDtypeStruct((n, m), jnp.float32),
  )(p, w1, w2)


def _add_pair(q, bm):
  _, n, m = q.shape

  def body(q_ref, o_ref):
    o_ref[...] = q_ref[0] + q_ref[1]

  return pl.pallas_call(
      body,
      grid=(n // bm,),
      in_specs=[pl.BlockSpec((2, bm, m), lambda i: (0, i, 0))],
      out_specs=pl.BlockSpec((bm, m), lambda i: (i, 0)),
      out_shape=jax.ShapeDtypeStruct((n, m), jnp.float32),
  )(q)


def _add_relu(p, bm):
  _, n, k = p.shape

  def body(p_ref, o_ref):
    o_ref[...] = jnp.maximum(p_ref[0] + p_ref[1], 0.0)

  return pl.pallas_call(
      body,
      grid=(n // bm,),
      in_specs=[pl.BlockSpec((2, bm, k), lambda i: (0, i, 0))],
      out_specs=pl.BlockSpec((bm, k), lambda i: (i, 0)),
      out_shape=jax.ShapeDtypeStruct((n, k), jnp.float32),
  )(p)


def _add_mm(q, w, bm):
  _, n, k = q.shape
  m = w.shape[1]

  def body(q_ref, w_ref, o_ref):
    o_ref[...] = jnp.dot(q_ref[0] + q_ref[1], w_ref[...],
                         preferred_element_type=jnp.float32)

  return pl.pallas_call(
      body,
      grid=(n // bm,),
      in_specs=[pl.BlockSpec((2, bm, k), lambda i: (0, i, 0)),
                pl.BlockSpec((k, m), lambda i: (0, 0))],
      out_specs=pl.BlockSpec((bm, m), lambda i: (i, 0)),
      out_shape=jax.ShapeDtypeStruct((n, m), jnp.float32),
  )(q, w)


def kernel(x, edge_index, edge_weight, W1, W2):
  n, in_dim = x.shape
  e = edge_weight.shape[0]
  hidden = W1.shape[1]
  out_dim = W2.shape[1]
  epw = e // _NW
  chunk = 80
  nchunk = epw // chunk
  ngrp = 5
  grp = nchunk // ngrp

  row = edge_index[0].reshape(_NW, ngrp, grp, chunk)
  col = edge_index[1].reshape(_NW, ngrp, grp, chunk)
  w = edge_weight.reshape(_NW, ngrp, grp * chunk)

  spmm1 = _make_spmm_bf16(n, e, in_dim)
  spmm2 = _make_spmm(n, e, out_dim, tc_tiling=False)

  p = spmm1(_interleave_bf16(x), row, col, w)
  h2 = _add_mm_relu_mm(p, W1, W2, 1000)
  q = spmm2(h2, row, col, w)
  return _add_pair(q, 1000)


# speedup vs baseline: 1.0709x; 1.0709x over previous
import dataclasses
import functools

import jax
import jax.numpy as jnp
from jax import lax
from jax.experimental import pallas as pl
from jax.experimental.pallas import tpu as pltpu
from jax.experimental.pallas import tpu_sc as plsc

_NC = 2
_NS = 16
_NW = _NC * _NS
_LANES = 16


def _make_spmm(n, e, d, tc_tiling=True):
  epw = e // _NW
  chunk = 80
  nchunk = epw // chunk
  ngrp = 5
  grp = nchunk // ngrp
  rps = 624
  tail = n - rps * _NS
  zrows = 16
  assert epw % chunk == 0 and rps % zrows == 0 and 0 <= tail <= zrows
  assert nchunk % ngrp == 0
  mesh = plsc.VectorSubcoreMesh(core_axis_name="c", subcore_axis_name="s")
  cp = pltpu.CompilerParams()
  if "needs_layout_passes" in pltpu.CompilerParams.__dataclass_fields__:
    cp = dataclasses.replace(cp, needs_layout_passes=False)
  if not tc_tiling:
    cp = dataclasses.replace(cp, use_tc_tiling_on_sc=False)

  @functools.partial(
      pl.kernel,
      compiler_params=cp,
      out_type=jax.ShapeDtypeStruct((_NC, n, d), jnp.float32),
      mesh=mesh,
      scratch_types=[
          pltpu.VMEM((grp, chunk), jnp.int32),
          pltpu.VMEM((grp, chunk), jnp.int32),
          pltpu.VMEM((grp * chunk,), jnp.float32),
          pltpu.VMEM((chunk, d), jnp.float32),
          pltpu.VMEM((chunk, d), jnp.float32),
          pltpu.VMEM((chunk, d), jnp.float32),
          pltpu.VMEM_SHARED((n, d), jnp.float32),
          [pltpu.SemaphoreType.DMA] * 3,
          [pltpu.SemaphoreType.DMA] * 3,
      ],
  )
  def spmm(h_hbm, row_hbm, col_hbm, w_hbm, out_hbm,
           row_v, col_v, w_v, buf, buf1, buf2, acc, gsems, ssems):
    cid = lax.axis_index("c")
    sid = lax.axis_index("s")
    wid = sid * _NC + cid

    zero = jnp.zeros((_LANES,), jnp.float32)

    @pl.loop(0, zrows)
    def _(i):
      for k in range(d // _LANES):
        buf[i, pl.ds(k * _LANES, _LANES)] = zero

    @pl.loop(0, rps // zrows)
    def _(i):
      pltpu.sync_copy(buf.at[pl.ds(0, zrows)],
                      acc.at[pl.ds(sid * rps + i * zrows, zrows)])

    @pl.when(sid == _NS - 1)
    def _():
      pltpu.sync_copy(buf.at[pl.ds(0, tail)],
                      acc.at[pl.ds(_NS * rps, tail)])

    plsc.subcore_barrier()

    def scale(bufref, j):
      dnums = lax.GatherDimensionNumbers(
          offset_dims=(), collapsed_slice_dims=(0,), start_index_map=(0,))

      @plsc.parallel_loop(0, chunk, step=_LANES, unroll=2)
      def _(e0):
        w16 = w_v[pl.ds(j * chunk + e0, _LANES)]
        for r in range(_LANES):
          wreg = lax.gather(
              w16, jnp.full((_LANES, 1), r, jnp.int32), dnums, (1,),
              mode=lax.GatherScatterMode.PROMISE_IN_BOUNDS)
          for k in range(d // _LANES):
            sl = (e0 + r, pl.ds(k * _LANES, _LANES))
            bufref[sl] = bufref[sl] * wreg

    bufs = (buf, buf1, buf2)

    def wait_gather(b, j):
      pltpu.make_async_copy(h_hbm.at[col_v.at[j]], bufs[b], gsems[b]).wait()

    def wait_scatter(b):
      pltpu.make_async_copy(bufs[b], acc.at[row_v.at[0]], ssems[b]).wait()

    @pl.loop(0, ngrp)
    def _(g):
      @pl.when(g > 0)
      def _():
        for b in range(3):
          wait_scatter(b)

      pltpu.sync_copy(row_hbm.at[wid, g], row_v)
      pltpu.sync_copy(col_hbm.at[wid, g], col_v)
      pltpu.sync_copy(w_hbm.at[wid, g], w_v)

      pltpu.async_copy(h_hbm.at[col_v.at[0]], buf, gsems[0])
      pltpu.async_copy(h_hbm.at[col_v.at[1]], buf1, gsems[1])

      @pl.loop(0, (grp + 2) // 3)
      def _(i):
        for b in range(3):
          j = 3 * i + b

          @pl.when(j < grp)
          def _():
            wait_gather(b, j)
            b2 = (b + 2) % 3

            @pl.when(j + 2 < grp)
            def _():
              @pl.when(j >= 1)
              def _():
                wait_scatter(b2)
              pltpu.async_copy(h_hbm.at[col_v.at[j + 2]], bufs[b2], gsems[b2])

            scale(bufs[b], j)
            pltpu.async_copy(bufs[b], acc.at[row_v.at[j]], ssems[b], add=True)

    for b in range(3):
      wait_scatter(b)

    plsc.subcore_barrier()

    pltpu.sync_copy(acc.at[pl.ds(sid * rps, rps)],
                    out_hbm.at[cid, pl.ds(sid * rps, rps)])

    @pl.when(sid == _NS - 1)
    def _():
      pltpu.sync_copy(acc.at[pl.ds(_NS * rps, tail)],
                      out_hbm.at[cid, pl.ds(_NS * rps, tail)])

  return spmm


def _add_mm_relu_mm(p, w1, w2, bm):
  _, n, k = p.shape
  m = w2.shape[1]

  def body(p_ref, w1_ref, w2_ref, o_ref):
    t = jnp.dot(p_ref[0] + p_ref[1], w1_ref[...],
                preferred_element_type=jnp.float32)
    o_ref[...] = jnp.dot(jnp.maximum(t, 0.0), w2_ref[...],
                         preferred_element_type=jnp.float32)

  return pl.pallas_call(
      body,
      grid=(n // bm,),
      in_specs=[pl.BlockSpec((2, bm, k), lambda i: (0, i, 0)),
                pl.BlockSpec((k, w1.shape[1]), lambda i: (0, 0)),
                pl.BlockSpec((w2.shape[0], m), lambda i: (0, 0))],
      out_specs=pl.BlockSpec((bm, m), lambda i: (i, 0)),
      out_shape=jax.ShapeDtypeStruct((n, m), jnp.float32),
  )(p, w1, w2)


def _add_pair(q, bm):
  _, n, m = q.shape

  def body(q_ref, o_ref):
    o_ref[...] = q_ref[0] + q_ref[1]

  return pl.pallas_call(
      body,
      grid=(n // bm,),
      in_specs=[pl.BlockSpec((2, bm, m), lambda i: (0, i, 0))],
      out_specs=pl.BlockSpec((bm, m), lambda i: (i, 0)),
      out_shape=jax.ShapeDtypeStruct((n, m), jnp.float32),
  )(q)


def kernel(x, edge_index, edge_weight, W1, W2):
  n, in_dim = x.shape
  e = edge_weight.shape[0]
  hidden = W1.shape[1]
  out_dim = W2.shape[1]
  epw = e // _NW
  chunk = 80
  nchunk = epw // chunk
  ngrp = 5
  grp = nchunk // ngrp

  row = edge_index[0].reshape(_NW, ngrp, grp, chunk)
  col = edge_index[1].reshape(_NW, ngrp, grp, chunk)
  w = edge_weight.reshape(_NW, ngrp, grp * chunk)

  spmm1 = _make_spmm(n, e, in_dim)
  spmm2 = _make_spmm(n, e, out_dim, tc_tiling=False)

  p = spmm1(x, row, col, w)
  h2 = _add_mm_relu_mm(p, W1, W2, 1000)
  q = spmm2(h2, row, col, w)
  return _add_pair(q, 1000)
